# baseline (speedup 1.0000x reference)
import math

import jax
import jax.numpy as jnp
from jax import lax
from jax.experimental import pallas as pl
from jax.experimental.pallas import tpu as pltpu

_TILE = 512
_UNROLL = 8
_SCALE_TILE = 1024


def _round_up(x: int, m: int) -> int:
    return ((x + m - 1) // m) * m


def _scale_body(scale: float):
    def _body(tbl_ref, out_ref):
        rows = pltpu.bitcast(tbl_ref[...], jnp.bfloat16)
        scaled = (rows.astype(jnp.float32) * scale).astype(jnp.bfloat16)
        out_ref[...] = pltpu.bitcast(scaled, jnp.int32)

    return _body


def _gather_body(tile: int, unroll: int):
    def _body(ids_ref, tbl_ref, out_ref):
        def gather_one(mi):
            tok = ids_ref[0, 0, mi]
            out_ref[mi] = tbl_ref[tok]

        @pl.loop(0, tile // unroll)
        def _(k):
            for j in range(unroll):
                gather_one(k * unroll + j)

    return _body


def kernel(tokens, emb_table):
    vocab, emb = emb_table.shape
    assert emb % 256 == 0, "feature dim must pack into i32 lane pairs"
    scale = float(math.sqrt(emb))
    half = emb // 2

    flat = jnp.clip(tokens.reshape(-1).astype(jnp.int32), 0, vocab - 1)
    n_tok = int(flat.shape[0])

    tile = min(_TILE, _round_up(n_tok, _UNROLL))
    n_pad = _round_up(n_tok, tile)
    ids = jnp.pad(flat, (0, n_pad - n_tok))
    n_blocks = n_pad // tile
    ids3d = ids.reshape(n_blocks, 1, tile)

    lo = lax.bitcast_convert_type(emb_table[:, 0::2], jnp.uint16)
    hi = lax.bitcast_convert_type(emb_table[:, 1::2], jnp.uint16)
    tbl_i32 = (lo.astype(jnp.uint32)
               | (hi.astype(jnp.uint32) << 16)).astype(jnp.int32)

    v_tile = min(_SCALE_TILE, _round_up(vocab, 8))
    v_pad = _round_up(vocab, v_tile)
    if v_pad != vocab:
        tbl_i32 = jnp.pad(tbl_i32, ((0, v_pad - vocab), (0, 0)))
    tbl_scaled = pl.pallas_call(
        _scale_body(scale),
        out_shape=jax.ShapeDtypeStruct((v_pad, half), jnp.int32),
        grid=(v_pad // v_tile,),
        in_specs=[pl.BlockSpec((v_tile, half), lambda i: (i, 0))],
        out_specs=pl.BlockSpec((v_tile, half), lambda i: (i, 0)),
        compiler_params=pltpu.CompilerParams(
            dimension_semantics=("parallel",),
        ),
    )(tbl_i32)
    tbl3d = tbl_scaled.reshape(v_pad, 1, half)

    vmem_limit = int(2 * v_pad * half * 4
                     + 2 * tile * half * 4
                     + (4 << 20))
    out3d = pl.pallas_call(
        _gather_body(tile, _UNROLL),
        out_shape=jax.ShapeDtypeStruct((n_pad, 1, half), jnp.int32),
        grid=(n_blocks,),
        in_specs=[
            pl.BlockSpec((1, 1, tile), lambda i: (i, 0, 0),
                         memory_space=pltpu.SMEM),
            pl.BlockSpec((v_pad, 1, half), lambda i: (0, 0, 0)),
        ],
        out_specs=pl.BlockSpec((tile, 1, half), lambda i: (i, 0, 0)),
        compiler_params=pltpu.CompilerParams(
            dimension_semantics=("parallel",),
            vmem_limit_bytes=min(vmem_limit, 128 << 20),
        ),
    )(ids3d, tbl3d)

    out_bf16 = lax.bitcast_convert_type(out3d.reshape(n_pad, half),
                                        jnp.bfloat16)
    out_flat = out_bf16.reshape(n_pad, emb)
    return out_flat[:n_tok].reshape(tokens.shape + (emb,))


# speedup vs baseline: 1.4876x; 1.4876x over previous
import math

import jax
import jax.numpy as jnp
from jax import lax
from jax.experimental import pallas as pl
from jax.experimental.pallas import tpu as pltpu

_TILE = 512
_UNROLL = 8


def _round_up(x: int, m: int) -> int:
    return ((x + m - 1) // m) * m


def _make_body(tile: int, unroll: int, scale: float):
    def _body(ids_ref, tbl_ref, out_ref, scratch):
        def gather_one(mi):
            tok = ids_ref[0, 0, mi]
            scratch[mi] = tbl_ref[tok]

        @pl.loop(0, tile // unroll)
        def _(k):
            for j in range(unroll):
                gather_one(k * unroll + j)

        rows = pltpu.bitcast(scratch[...], jnp.bfloat16)
        out_ref[...] = (rows.astype(jnp.float32) * scale).astype(out_ref.dtype)

    return _body


def kernel(tokens, emb_table):
    vocab, emb = emb_table.shape
    assert emb % 256 == 0, "feature dim must split into two 128-lane halves"
    scale = float(math.sqrt(emb))
    out_dtype = emb_table.dtype
    half = emb // 2

    flat = jnp.clip(tokens.reshape(-1).astype(jnp.int32), 0, vocab - 1)
    n_tok = int(flat.shape[0])

    tile = min(_TILE, _round_up(n_tok, _UNROLL))
    n_pad = _round_up(n_tok, tile)
    ids = jnp.pad(flat, (0, n_pad - n_tok))
    n_blocks = n_pad // tile
    ids3d = ids.reshape(n_blocks, 1, tile)

    lo = lax.bitcast_convert_type(emb_table[:, :half], jnp.uint16)
    hi = lax.bitcast_convert_type(emb_table[:, half:], jnp.uint16)
    tbl_i32 = (lo.astype(jnp.uint32)
               | (hi.astype(jnp.uint32) << 16)).astype(jnp.int32)
    tbl_i32 = tbl_i32.reshape(vocab, 1, half)

    vmem_limit = int(2 * vocab * half * 4
                     + 2 * tile * emb * 2
                     + tile * half * 4
                     + (4 << 20))

    out3d = pl.pallas_call(
        _make_body(tile, _UNROLL, scale),
        out_shape=jax.ShapeDtypeStruct((n_pad, 2, half), out_dtype),
        grid=(n_blocks,),
        in_specs=[
            pl.BlockSpec((1, 1, tile), lambda i: (i, 0, 0),
                         memory_space=pltpu.SMEM),
            pl.BlockSpec((vocab, 1, half), lambda i: (0, 0, 0)),
        ],
        out_specs=pl.BlockSpec((tile, 2, half), lambda i: (i, 0, 0)),
        scratch_shapes=[pltpu.VMEM((tile, 1, half), jnp.int32)],
        compiler_params=pltpu.CompilerParams(
            dimension_semantics=("parallel",),
            vmem_limit_bytes=min(vmem_limit, 128 << 20),
        ),
    )(ids3d, tbl_i32)

    out_flat = out3d.reshape(n_pad, emb)
    return out_flat[:n_tok].reshape(tokens.shape + (emb,))


# speedup vs baseline: 3.1075x; 2.0889x over previous
import math

import jax
import jax.numpy as jnp
from jax import lax
from jax.experimental import pallas as pl
from jax.experimental.pallas import tpu as pltpu

_TILE = 512
_UNROLL = 16


def _round_up(x: int, m: int) -> int:
    return ((x + m - 1) // m) * m


def _make_body(tile: int, unroll: int, scale: float):
    def _body(ids_ref, tbl_ref, out_ref, scratch):
        def gather_one(mi):
            tok = ids_ref[0, 0, mi]
            base = pl.multiple_of((tok >> 3) << 3, 8)
            chunk = tbl_ref[pl.ds(base, 8), :].astype(jnp.float32)
            rolled = pltpu.roll(chunk, -(tok & 7), axis=0)
            scratch[pl.ds(mi, 1), :] = rolled[0:1, :]

        @pl.loop(0, tile // unroll)
        def _(k):
            for j in range(unroll):
                gather_one(k * unroll + j)

        out_ref[...] = (scratch[...] * scale).astype(out_ref.dtype)

    return _body


def kernel(tokens, emb_table):
    vocab, emb = emb_table.shape
    scale = float(math.sqrt(emb))
    out_dtype = emb_table.dtype

    flat = jnp.clip(tokens.reshape(-1).astype(jnp.int32), 0, vocab - 1)
    n_tok = int(flat.shape[0])

    tile = min(_TILE, _round_up(n_tok, _UNROLL))
    n_pad = _round_up(n_tok, tile)
    ids = jnp.pad(flat, (0, n_pad - n_tok))
    n_blocks = n_pad // tile
    ids3d = ids.reshape(n_blocks, 1, tile)

    v_pad = _round_up(vocab, 8)
    tbl = emb_table
    if v_pad != vocab:
        tbl = jnp.pad(emb_table, ((0, v_pad - vocab), (0, 0)))

    itemsize = jnp.dtype(out_dtype).itemsize
    vmem_limit = int(2 * v_pad * emb * itemsize
                     + 2 * tile * emb * itemsize
                     + tile * emb * 4
                     + (4 << 20))

    out_flat = pl.pallas_call(
        _make_body(tile, _UNROLL, scale),
        out_shape=jax.ShapeDtypeStruct((n_pad, emb), out_dtype),
        grid=(n_blocks,),
        in_specs=[
            pl.BlockSpec((1, 1, tile), lambda i: (i, 0, 0),
                         memory_space=pltpu.SMEM),
            pl.BlockSpec((v_pad, emb), lambda i: (0, 0)),
        ],
        out_specs=pl.BlockSpec((tile, emb), lambda i: (i, 0)),
        scratch_shapes=[pltpu.VMEM((tile, emb), jnp.float32)],
        compiler_params=pltpu.CompilerParams(
            dimension_semantics=("parallel",),
            vmem_limit_bytes=min(vmem_limit, 128 << 20),
        ),
    )(ids3d, tbl)

    return out_flat[:n_tok].reshape(tokens.shape + (emb,))


# speedup vs baseline: 3.1572x; 1.0160x over previous
import math

import jax
import jax.numpy as jnp
from jax import lax
from jax.experimental import pallas as pl
from jax.experimental.pallas import tpu as pltpu

_TILE = 2048
_UNROLL = 16


def _round_up(x: int, m: int) -> int:
    return ((x + m - 1) // m) * m


def _make_body(tile: int, unroll: int, scale: float):
    def _body(ids_ref, tbl_ref, out_ref, scratch):
        def gather_one(mi):
            tok = ids_ref[0, 0, mi]
            base = pl.multiple_of((tok >> 3) << 3, 8)
            chunk = tbl_ref[pl.ds(base, 8), :].astype(jnp.float32)
            rolled = pltpu.roll(chunk, -(tok & 7), axis=0)
            scratch[pl.ds(mi, 1), :] = rolled[0:1, :]

        @pl.loop(0, tile // unroll)
        def _(k):
            for j in range(unroll):
                gather_one(k * unroll + j)

        out_ref[...] = (scratch[...] * scale).astype(out_ref.dtype)

    return _body


def kernel(tokens, emb_table):
    vocab, emb = emb_table.shape
    scale = float(math.sqrt(emb))
    out_dtype = emb_table.dtype

    flat = jnp.clip(tokens.reshape(-1).astype(jnp.int32), 0, vocab - 1)
    n_tok = int(flat.shape[0])

    tile = min(_TILE, _round_up(n_tok, _UNROLL))
    n_pad = _round_up(n_tok, tile)
    ids = jnp.pad(flat, (0, n_pad - n_tok))
    n_blocks = n_pad // tile
    ids3d = ids.reshape(n_blocks, 1, tile)

    v_pad = _round_up(vocab, 8)
    tbl = emb_table
    if v_pad != vocab:
        tbl = jnp.pad(emb_table, ((0, v_pad - vocab), (0, 0)))

    itemsize = jnp.dtype(out_dtype).itemsize
    vmem_limit = int(2 * v_pad * emb * itemsize
                     + 2 * tile * emb * itemsize
                     + tile * emb * 4
                     + (4 << 20))

    out_flat = pl.pallas_call(
        _make_body(tile, _UNROLL, scale),
        out_shape=jax.ShapeDtypeStruct((n_pad, emb), out_dtype),
        grid=(n_blocks,),
        in_specs=[
            pl.BlockSpec((1, 1, tile), lambda i: (i, 0, 0),
                         memory_space=pltpu.SMEM),
            pl.BlockSpec((v_pad, emb), lambda i: (0, 0)),
        ],
        out_specs=pl.BlockSpec((tile, emb), lambda i: (i, 0)),
        scratch_shapes=[pltpu.VMEM((tile, emb), jnp.float32)],
        compiler_params=pltpu.CompilerParams(
            dimension_semantics=("parallel",),
            vmem_limit_bytes=min(vmem_limit, 128 << 20),
        ),
    )(ids3d, tbl)

    return out_flat[:n_tok].reshape(tokens.shape + (emb,))


# speedup vs baseline: 3.3434x; 1.0590x over previous
import math

import jax
import jax.numpy as jnp
from jax import lax
from jax.experimental import pallas as pl
from jax.experimental.pallas import tpu as pltpu

_TILE = 2048
_UNROLL = 32


def _round_up(x: int, m: int) -> int:
    return ((x + m - 1) // m) * m


def _make_body(tile: int, unroll: int, scale: float):
    def _body(ids_ref, tbl_ref, out_ref, scratch):
        def gather_one(mi):
            tok = ids_ref[0, 0, mi]
            base = pl.multiple_of((tok >> 3) << 3, 8)
            chunk = tbl_ref[pl.ds(base, 8), :].astype(jnp.float32)
            rolled = pltpu.roll(chunk, -(tok & 7), axis=0)
            scratch[pl.ds(mi, 1), :] = rolled[0:1, :]

        @pl.loop(0, tile // unroll)
        def _(k):
            for j in range(unroll):
                gather_one(k * unroll + j)

        out_ref[...] = (scratch[...] * scale).astype(out_ref.dtype)

    return _body


def kernel(tokens, emb_table):
    vocab, emb = emb_table.shape
    scale = float(math.sqrt(emb))
    out_dtype = emb_table.dtype

    flat = jnp.clip(tokens.reshape(-1).astype(jnp.int32), 0, vocab - 1)
    n_tok = int(flat.shape[0])

    tile = min(_TILE, _round_up(n_tok, _UNROLL))
    n_pad = _round_up(n_tok, tile)
    ids = jnp.pad(flat, (0, n_pad - n_tok))
    n_blocks = n_pad // tile
    ids3d = ids.reshape(n_blocks, 1, tile)

    v_pad = _round_up(vocab, 8)
    tbl = emb_table
    if v_pad != vocab:
        tbl = jnp.pad(emb_table, ((0, v_pad - vocab), (0, 0)))

    itemsize = jnp.dtype(out_dtype).itemsize
    vmem_limit = int(2 * v_pad * emb * itemsize
                     + 2 * tile * emb * itemsize
                     + tile * emb * 4
                     + (4 << 20))

    out_flat = pl.pallas_call(
        _make_body(tile, _UNROLL, scale),
        out_shape=jax.ShapeDtypeStruct((n_pad, emb), out_dtype),
        grid=(n_blocks,),
        in_specs=[
            pl.BlockSpec((1, 1, tile), lambda i: (i, 0, 0),
                         memory_space=pltpu.SMEM),
            pl.BlockSpec((v_pad, emb), lambda i: (0, 0)),
        ],
        out_specs=pl.BlockSpec((tile, emb), lambda i: (i, 0)),
        scratch_shapes=[pltpu.VMEM((tile, emb), jnp.float32)],
        compiler_params=pltpu.CompilerParams(
            dimension_semantics=("parallel",),
            vmem_limit_bytes=min(vmem_limit, 128 << 20),
        ),
    )(ids3d, tbl)

    return out_flat[:n_tok].reshape(tokens.shape + (emb,))


# speedup vs baseline: 6.3222x; 1.8909x over previous
import math

import jax
import jax.numpy as jnp
from jax import lax
from jax.experimental import pallas as pl
from jax.experimental.pallas import tpu as pltpu

_TILE = 2048
_GUNROLL = 128


def _round_up(x: int, m: int) -> int:
    return ((x + m - 1) // m) * m


def _make_body(tile: int, gunroll: int, scale: float):
    def _body(ids_ref, tbl_ref, sel_ref, out_ref):
        def gather_group(g):
            row0 = g * 8
            chunks = []
            for j in range(8):
                tok = ids_ref[0, 0, row0 + j]
                base = pl.multiple_of((tok >> 3) << 3, 8)
                chunks.append(tbl_ref[pl.ds(base, 8), :])
            cand = jnp.concatenate(chunks, axis=0)
            sel = sel_ref[pl.ds(row0, 8), :]
            picked = lax.dot_general(
                sel, cand, (((1,), (0,)), ((), ())),
                preferred_element_type=jnp.float32)
            out_ref[pl.ds(row0, 8), :] = (picked * scale).astype(out_ref.dtype)

        @pl.loop(0, tile // (8 * gunroll))
        def _(k):
            for j in range(gunroll):
                gather_group(k * gunroll + j)

    return _body


def kernel(tokens, emb_table):
    vocab, emb = emb_table.shape
    scale = float(math.sqrt(emb))
    out_dtype = emb_table.dtype

    flat = jnp.clip(tokens.reshape(-1).astype(jnp.int32), 0, vocab - 1)
    n_tok = int(flat.shape[0])

    tile = min(_TILE, _round_up(n_tok, 8 * _GUNROLL))
    n_pad = _round_up(n_tok, tile)
    ids = jnp.pad(flat, (0, n_pad - n_tok))
    n_blocks = n_pad // tile
    ids3d = ids.reshape(n_blocks, 1, tile)

    tsel = (jnp.arange(n_pad, dtype=jnp.int32) % 8) * 8 + (ids & 7)
    sel = (tsel[:, None] == jnp.arange(64, dtype=jnp.int32)[None, :]
           ).astype(emb_table.dtype)

    v_pad = _round_up(vocab, 8)
    tbl = emb_table
    if v_pad != vocab:
        tbl = jnp.pad(emb_table, ((0, v_pad - vocab), (0, 0)))

    itemsize = jnp.dtype(out_dtype).itemsize
    vmem_limit = int(2 * v_pad * emb * itemsize
                     + 2 * tile * emb * itemsize
                     + 2 * tile * 64 * itemsize
                     + (4 << 20))

    out_flat = pl.pallas_call(
        _make_body(tile, _GUNROLL, scale),
        out_shape=jax.ShapeDtypeStruct((n_pad, emb), out_dtype),
        grid=(n_blocks,),
        in_specs=[
            pl.BlockSpec((1, 1, tile), lambda i: (i, 0, 0),
                         memory_space=pltpu.SMEM),
            pl.BlockSpec((v_pad, emb), lambda i: (0, 0)),
            pl.BlockSpec((tile, 64), lambda i: (i, 0)),
        ],
        out_specs=pl.BlockSpec((tile, emb), lambda i: (i, 0)),
        compiler_params=pltpu.CompilerParams(
            dimension_semantics=("parallel",),
            vmem_limit_bytes=min(vmem_limit, 128 << 20),
        ),
    )(ids3d, tbl, sel)

    return out_flat[:n_tok].reshape(tokens.shape + (emb,))


# speedup vs baseline: 6.3340x; 1.0019x over previous
import math

import jax
import jax.numpy as jnp
from jax import lax
from jax.experimental import pallas as pl
from jax.experimental.pallas import tpu as pltpu

_TILE = 4096
_GUNROLL = 128


def _round_up(x: int, m: int) -> int:
    return ((x + m - 1) // m) * m


def _make_body(tile: int, gunroll: int, scale: float):
    def _body(ids_ref, tbl_ref, sel_ref, out_ref):
        def gather_group(g):
            row0 = g * 8
            chunks = []
            for j in range(8):
                tok = ids_ref[0, 0, row0 + j]
                base = pl.multiple_of((tok >> 3) << 3, 8)
                chunks.append(tbl_ref[pl.ds(base, 8), :])
            cand = jnp.concatenate(chunks, axis=0)
            sel = sel_ref[pl.ds(row0, 8), :]
            picked = lax.dot_general(
                sel, cand, (((1,), (0,)), ((), ())),
                preferred_element_type=jnp.float32)
            out_ref[pl.ds(row0, 8), :] = (picked * scale).astype(out_ref.dtype)

        @pl.loop(0, tile // (8 * gunroll))
        def _(k):
            for j in range(gunroll):
                gather_group(k * gunroll + j)

    return _body


def kernel(tokens, emb_table):
    vocab, emb = emb_table.shape
    scale = float(math.sqrt(emb))
    out_dtype = emb_table.dtype

    flat = jnp.clip(tokens.reshape(-1).astype(jnp.int32), 0, vocab - 1)
    n_tok = int(flat.shape[0])

    tile = min(_TILE, _round_up(n_tok, 8 * _GUNROLL))
    n_pad = _round_up(n_tok, tile)
    ids = jnp.pad(flat, (0, n_pad - n_tok))
    n_blocks = n_pad // tile
    ids3d = ids.reshape(n_blocks, 1, tile)

    tsel = (jnp.arange(n_pad, dtype=jnp.int32) % 8) * 8 + (ids & 7)
    sel = (tsel[:, None] == jnp.arange(64, dtype=jnp.int32)[None, :]
           ).astype(emb_table.dtype)

    v_pad = _round_up(vocab, 8)
    tbl = emb_table
    if v_pad != vocab:
        tbl = jnp.pad(emb_table, ((0, v_pad - vocab), (0, 0)))

    itemsize = jnp.dtype(out_dtype).itemsize
    vmem_limit = int(2 * v_pad * emb * itemsize
                     + 2 * tile * emb * itemsize
                     + 2 * tile * 64 * itemsize
                     + (4 << 20))

    out_flat = pl.pallas_call(
        _make_body(tile, _GUNROLL, scale),
        out_shape=jax.ShapeDtypeStruct((n_pad, emb), out_dtype),
        grid=(n_blocks,),
        in_specs=[
            pl.BlockSpec((1, 1, tile), lambda i: (i, 0, 0),
                         memory_space=pltpu.SMEM),
            pl.BlockSpec((v_pad, emb), lambda i: (0, 0)),
            pl.BlockSpec((tile, 64), lambda i: (i, 0)),
        ],
        out_specs=pl.BlockSpec((tile, emb), lambda i: (i, 0)),
        compiler_params=pltpu.CompilerParams(
            dimension_semantics=("parallel",),
            vmem_limit_bytes=min(vmem_limit, 128 << 20),
        ),
    )(ids3d, tbl, sel)

    return out_flat[:n_tok].reshape(tokens.shape + (emb,))
